# baseline (speedup 1.0000x reference)
import functools

import jax
import jax.numpy as jnp
from jax import lax
from jax.experimental import pallas as pl
from jax.experimental.pallas import tpu as pltpu
from jax.experimental.pallas import tpu_sc as plsc

_B, _T, _N, _D = 128, 64, 20, 7
_BT = _B * _T
_ROWS = _B * _T * _N
_CLS_W, _REG_W = 10.0, 8.0
_CLS_CNT = _B * (_T - 1) * _N
_REG_CNT = _B * (_T - 1) * _D

_NC, _NS = 2, 16
_NW = _NC * _NS
_CH = _BT // _NW
_HALF = _CH // 2


def _sgn(v):
    return jnp.where(v >= 0, 1.0, -1.0)


def _loss0(x):
    p = 1.0 / (1.0 + jnp.exp(-x))
    bce0 = jnp.maximum(x, 0.0) + jnp.log1p(jnp.exp(-jnp.abs(x)))
    return bce0 * (0.75 * p * p)


def _encode_argmin_body(ego_ref, cls0_ref, cls1_ref, cls2_ref, anchor_ref,
                        gt_ref, idx_ref, sums_ref):
    m00 = ego_ref[0]
    m01 = ego_ref[1]
    m02 = ego_ref[2]
    tx = ego_ref[3]
    m10 = ego_ref[4]
    m11 = ego_ref[5]
    m12 = ego_ref[6]
    ty = ego_ref[7]
    m20 = ego_ref[8]
    m21 = ego_ref[9]
    m22 = ego_ref[10]
    tz = ego_ref[11]
    eps = 1e-12
    qw = 0.5 * jnp.sqrt(jnp.maximum(1.0 + m00 + m11 + m22, eps))
    qx = _sgn(m21 - m12) * 0.5 * jnp.sqrt(jnp.maximum(1.0 + m00 - m11 - m22, eps))
    qy = _sgn(m02 - m20) * 0.5 * jnp.sqrt(jnp.maximum(1.0 + m11 - m00 - m22, eps))
    qz = _sgn(m10 - m01) * 0.5 * jnp.sqrt(jnp.maximum(1.0 + m22 - m00 - m11, eps))
    comps = (tx, ty, tz, qw, qx, qy, qz)
    for c in range(_D):
        gt_ref[c] = comps[c]
    bestd = jnp.full((_B, _T), jnp.inf, jnp.float32)
    bestn = jnp.zeros((_B, _T), jnp.int32)
    for n in range(_N):
        ax = anchor_ref[n, 0]
        ay = anchor_ref[n, 1]
        az = anchor_ref[n, 2]
        d = jnp.sqrt((tx - ax) ** 2 + (ty - ay) ** 2 + (tz - az) ** 2)
        upd = d < bestd
        bestd = jnp.where(upd, d, bestd)
        bestn = jnp.where(upd, n, bestn)
    bi = lax.broadcasted_iota(jnp.int32, (_B, _T), 0)
    ti = lax.broadcasted_iota(jnp.int32, (_B, _T), 1)
    idx_ref[...] = (bi * _T + ti) * _N + bestn
    lane = lax.broadcasted_iota(jnp.int32, (_B, _T * _N), 1)
    tmask = lane >= _N
    for l, ref in enumerate((cls0_ref, cls1_ref, cls2_ref)):
        x = ref[...]
        sums_ref[l] = jnp.sum(jnp.where(tmask, _loss0(x), 0.0))
    sums_ref[3] = 0.0


def _sc_gather_body(reg0, reg1, reg2, cls0, cls1, cls2, idx_hbm, gt_hbm,
                    regp_out, clsg_out, *s):
    idx_a, idx_b = s[0], s[1]
    idx7 = [[s[2 + d * 2 + h] for h in range(2)] for d in range(_D)]
    gtv = [s[16 + d] for d in range(_D)]
    gbuf = (s[23], s[24])
    clsb = (s[25], s[26], s[27])
    accv = s[28]
    sems = (s[29], s[30])
    semc = s[31]
    regs = (reg0, reg1, reg2)
    wid = lax.axis_index("s") * _NC + lax.axis_index("c")
    base = wid * _CH
    pltpu.sync_copy(idx_hbm.at[pl.ds(base, _HALF)], idx_a)
    pltpu.sync_copy(idx_hbm.at[pl.ds(base + _HALF, _HALF)], idx_b)
    cls_copies = []
    for l, clst in enumerate((cls0, cls1, cls2)):
        for h, iv in enumerate((idx_a, idx_b)):
            cls_copies.append(pltpu.async_copy(
                clst.at[iv], clsb[l].at[pl.ds(h * _HALF, _HALF)], semc))
    for d in range(_D):
        pltpu.sync_copy(gt_hbm.at[pl.ds(d * _BT + base, _CH)], gtv[d])
    for h, iv in enumerate((idx_a, idx_b)):
        for i in range(_HALF // 16):
            v7 = iv[pl.ds(i * 16, 16)] * _D
            for d in range(_D):
                idx7[d][h][pl.ds(i * 16, 16)] = v7 + d
    steps = [(l, d) for l in range(3) for d in range(_D)]

    def fire(si):
        l, d = steps[si]
        bb = gbuf[si % 2]
        return [pltpu.async_copy(regs[l].at[idx7[d][h]],
                                 bb.at[pl.ds(h * _HALF, _HALF)], sems[si % 2])
                for h in range(2)]

    pend = fire(0)
    iota = lax.iota(jnp.int32, 16)
    mask0 = jnp.where(iota != 0, 1.0, 0.0).astype(jnp.float32)
    acc = jnp.zeros((16,), jnp.float32)
    for si, (l, d) in enumerate(steps):
        nxt = fire(si + 1) if si + 1 < len(steps) else None
        for cp in pend:
            cp.wait()
        bb = gbuf[si % 2]
        for i in range(_CH // 16):
            diff = jnp.abs(bb[pl.ds(i * 16, 16)] - gtv[d][pl.ds(i * 16, 16)])
            if i % 4 == 0:
                diff = diff * mask0
            acc = acc + diff
        pend = nxt
        if d == _D - 1:
            accv[...] = acc
            pltpu.sync_copy(accv, regp_out.at[pl.ds((l * _NW + wid) * 16, 16)])
            acc = jnp.zeros((16,), jnp.float32)
    for cp in cls_copies:
        cp.wait()
    for l in range(3):
        pltpu.sync_copy(clsb[l], clsg_out.at[pl.ds(l * _BT + base, _CH)])


def _finalize_body(regp_ref, clsg_ref, sums_ref, reg_ref, cls_ref, tot_ref):
    lane = lax.broadcasted_iota(jnp.int32, (_B, _T), 1)
    m = lane >= 1
    total = jnp.float32(0.0)
    for l in range(3):
        x = clsg_ref[pl.ds(l * _B, _B), :]
        p = 1.0 / (1.0 + jnp.exp(-x))
        bce0 = jnp.maximum(x, 0.0) + jnp.log1p(jnp.exp(-jnp.abs(x)))
        bce1 = bce0 - x
        corr = bce1 * (0.25 * (1.0 - p) * (1.0 - p)) - bce0 * (0.75 * p * p)
        corr_sum = jnp.sum(jnp.where(m, corr, 0.0))
        cls_l = (sums_ref[l] + corr_sum) * (1.0 / _CLS_CNT)
        reg_l = jnp.sum(regp_ref[l]) * (1.0 / _REG_CNT)
        reg_ref[l] = reg_l
        cls_ref[l] = cls_l
        total = total + cls_l * _CLS_W + reg_l * _REG_W
    reg_ref[3] = 0.0
    cls_ref[3] = 0.0
    tot_ref[0] = total


def _run_encode_argmin(ego_cm, cls0, cls1, cls2, anchor):
    return pl.pallas_call(
        _encode_argmin_body,
        out_shape=(
            jax.ShapeDtypeStruct((_D, _B, _T), jnp.float32),
            jax.ShapeDtypeStruct((_B, _T), jnp.int32),
            jax.ShapeDtypeStruct((4,), jnp.float32),
        ),
        in_specs=[
            pl.BlockSpec(memory_space=pltpu.VMEM),
            pl.BlockSpec(memory_space=pltpu.VMEM),
            pl.BlockSpec(memory_space=pltpu.VMEM),
            pl.BlockSpec(memory_space=pltpu.VMEM),
            pl.BlockSpec(memory_space=pltpu.SMEM),
        ],
        out_specs=(
            pl.BlockSpec(memory_space=pltpu.VMEM),
            pl.BlockSpec(memory_space=pltpu.VMEM),
            pl.BlockSpec(memory_space=pltpu.SMEM),
        ),
    )(ego_cm, cls0, cls1, cls2, anchor)


def _run_sc_gather(reg0, reg1, reg2, cls0, cls1, cls2, idx, gt):
    mesh = plsc.VectorSubcoreMesh(core_axis_name="c", subcore_axis_name="s",
                                  num_cores=_NC)
    f32 = jnp.float32
    kern = functools.partial(
        pl.kernel,
        mesh=mesh,
        out_type=(
            jax.ShapeDtypeStruct((3 * _NW * 16,), f32),
            jax.ShapeDtypeStruct((3 * _BT,), f32),
        ),
        scratch_types=(
            [pltpu.VMEM((_HALF,), jnp.int32)] * 2
            + [pltpu.VMEM((_HALF,), jnp.int32)] * 14
            + [pltpu.VMEM((_CH,), f32)] * _D
            + [pltpu.VMEM((_CH,), f32)] * 2
            + [pltpu.VMEM((_CH,), f32)] * 3
            + [pltpu.VMEM((16,), f32)]
            + [pltpu.SemaphoreType.DMA] * 3
        ),
    )(_sc_gather_body)
    return kern(reg0, reg1, reg2, cls0, cls1, cls2, idx, gt)


def _run_finalize(regp, clsg, sums):
    return pl.pallas_call(
        _finalize_body,
        out_shape=(
            jax.ShapeDtypeStruct((4,), jnp.float32),
            jax.ShapeDtypeStruct((4,), jnp.float32),
            jax.ShapeDtypeStruct((1,), jnp.float32),
        ),
        in_specs=[
            pl.BlockSpec(memory_space=pltpu.VMEM),
            pl.BlockSpec(memory_space=pltpu.VMEM),
            pl.BlockSpec(memory_space=pltpu.SMEM),
        ],
        out_specs=(
            pl.BlockSpec(memory_space=pltpu.SMEM),
            pl.BlockSpec(memory_space=pltpu.SMEM),
            pl.BlockSpec(memory_space=pltpu.SMEM),
        ),
    )(regp, clsg, sums)


def kernel(diff_poses_reg_0, diff_poses_reg_1, diff_poses_reg_2,
           diff_poses_cls_0, diff_poses_cls_1, diff_poses_cls_2,
           ego_past_to_ego_curr, anchor):
    ego_cm = jnp.transpose(ego_past_to_ego_curr.reshape(_B, _T, 16), (2, 0, 1))
    cls_flat = tuple(c.reshape(_B, _T * _N)
                     for c in (diff_poses_cls_0, diff_poses_cls_1, diff_poses_cls_2))
    anchor2 = anchor.reshape(_N, 3)
    gt, idx, sums = _run_encode_argmin(ego_cm, *cls_flat, anchor2)

    reg_tabs = tuple(r.reshape(_ROWS * _D)
                     for r in (diff_poses_reg_0, diff_poses_reg_1, diff_poses_reg_2))
    cls_1d = tuple(c.reshape(_ROWS) for c in cls_flat)
    regp, clsg = _run_sc_gather(*reg_tabs, *cls_1d, idx.reshape(_BT),
                                gt.reshape(_D * _BT))

    reg4, cls4, tot = _run_finalize(regp.reshape(3, _NW, 16),
                                    clsg.reshape(3 * _B, _T), sums)
    return reg4[:3], cls4[:3], tot[0]


# speedup vs baseline: 12.9237x; 12.9237x over previous
import functools

import jax
import jax.numpy as jnp
from jax import lax
from jax.experimental import pallas as pl
from jax.experimental.pallas import tpu as pltpu
from jax.experimental.pallas import tpu_sc as plsc

_B, _T, _N, _D = 128, 64, 20, 7
_Q = _T * _B
_CLS_W, _REG_W = 10.0, 8.0
_CLS_CNT = _B * (_T - 1) * _N
_REG_CNT = _B * (_T - 1) * _D

_NC, _NS = 2, 16
_NW = _NC * _NS
_CH = _Q // _NW
_HALF = _CH // 2


def _sgn(v):
    return jnp.where(v >= 0, 1.0, -1.0)


def _encode_argmin_body(ego_ref, anchor_ref, gt_ref, idxpb_ref, bestn_ref):
    e = [ego_ref[:, k, :] for k in range(12)]
    m00, m01, m02, tx = e[0], e[1], e[2], e[3]
    m10, m11, m12, ty = e[4], e[5], e[6], e[7]
    m20, m21, m22, tz = e[8], e[9], e[10], e[11]
    eps = 1e-12
    qw = 0.5 * jnp.sqrt(jnp.maximum(1.0 + m00 + m11 + m22, eps))
    qx = _sgn(m21 - m12) * 0.5 * jnp.sqrt(jnp.maximum(1.0 + m00 - m11 - m22, eps))
    qy = _sgn(m02 - m20) * 0.5 * jnp.sqrt(jnp.maximum(1.0 + m11 - m00 - m22, eps))
    qz = _sgn(m10 - m01) * 0.5 * jnp.sqrt(jnp.maximum(1.0 + m22 - m00 - m11, eps))
    comps = (tx, ty, tz, qw, qx, qy, qz)
    for c in range(_D):
        gt_ref[c] = comps[c]
    bestd = jnp.full((_T, _B), jnp.inf, jnp.float32)
    bestn = jnp.zeros((_T, _B), jnp.int32)
    for n in range(_N):
        ax = anchor_ref[0, n]
        ay = anchor_ref[1, n]
        az = anchor_ref[2, n]
        d = (tx - ax) ** 2 + (ty - ay) ** 2 + (tz - az) ** 2
        upd = d < bestd
        bestd = jnp.where(upd, d, bestd)
        bestn = jnp.where(upd, n, bestn)
    qv = (lax.broadcasted_iota(jnp.int32, (_T, _B), 0) * _B
          + lax.broadcasted_iota(jnp.int32, (_T, _B), 1))
    idxpb_ref[...] = bestn * (_D * _Q) + qv
    bestn_ref[...] = bestn


def _dense_cls_body(cls0_ref, cls1_ref, cls2_ref, bestn_ref, sums_ref):
    n = pl.program_id(0)

    @pl.when(n == 0)
    def _init():
        for l in range(8):
            sums_ref[l] = 0.0

    tmask = lax.broadcasted_iota(jnp.int32, (1, _T, _B), 1) >= 1
    sel = tmask & (bestn_ref[...][None] == n)
    for l, ref in enumerate((cls0_ref, cls1_ref, cls2_ref)):
        x = ref[...]
        u = jnp.exp(-jnp.abs(x))
        r = 1.0 / (1.0 + u)
        p = jnp.where(x >= 0, r, u * r)
        sp = jnp.maximum(x, 0.0) + jnp.log1p(u)
        l0 = sp * (0.75 * p * p)
        omp = 1.0 - p
        corr = (sp - x) * (0.25 * omp * omp) - l0
        sums_ref[l] += jnp.sum(jnp.where(tmask, l0, 0.0))
        sums_ref[3 + l] += jnp.sum(jnp.where(sel, corr, 0.0))


def _sc_gather_body(reg0, reg1, reg2, idxpb_hbm, gt_hbm, regp_out, *s):
    ipb_a, ipb_b = s[0], s[1]
    idx7 = [[s[2 + d * 2 + h] for h in range(2)] for d in range(_D)]
    gtv = [s[16 + d] for d in range(_D)]
    gbuf = [s[23 + i] for i in range(21)]
    accv = s[44]
    semi = s[45]
    semg = s[46]
    sems = [s[47 + i] for i in range(21)]
    regs = (reg0, reg1, reg2)
    wid = lax.axis_index("s") * _NC + lax.axis_index("c")
    base = wid * _CH
    idx_copies = [
        pltpu.async_copy(idxpb_hbm.at[pl.ds(base, _HALF)], ipb_a, semi),
        pltpu.async_copy(idxpb_hbm.at[pl.ds(base + _HALF, _HALF)], ipb_b, semi),
    ]
    gt_copies = [
        pltpu.async_copy(gt_hbm.at[pl.ds(d * _Q + base, _CH)], gtv[d], semg)
        for d in range(_D)
    ]
    for cp in idx_copies:
        cp.wait()
    for h, iv in enumerate((ipb_a, ipb_b)):
        for i in range(_HALF // 16):
            v = iv[pl.ds(i * 16, 16)]
            for d in range(_D):
                idx7[d][h][pl.ds(i * 16, 16)] = v + (d * _Q)
    steps = [(l, d) for l in range(3) for d in range(_D)]
    pend = []
    for si, (l, d) in enumerate(steps):
        pend.append([pltpu.async_copy(regs[l].at[idx7[d][h]],
                                      gbuf[si].at[pl.ds(h * _HALF, _HALF)],
                                      sems[si])
                     for h in range(2)])
    for cp in gt_copies:
        cp.wait()
    zf = jnp.where(wid == 0, jnp.float32(0.0), jnp.float32(1.0))
    acc = jnp.zeros((16,), jnp.float32)
    for si, (l, d) in enumerate(steps):
        for cp in pend[si]:
            cp.wait()
        bb = gbuf[si]
        for i in range(_CH // 16):
            diff = jnp.abs(bb[pl.ds(i * 16, 16)] - gtv[d][pl.ds(i * 16, 16)])
            if i < 8:
                diff = diff * zf
            acc = acc + diff
        if d == _D - 1:
            accv[...] = acc
            pltpu.sync_copy(accv, regp_out.at[pl.ds((l * _NW + wid) * 16, 16)])
            acc = jnp.zeros((16,), jnp.float32)


def _finalize_body(regp_ref, sums_ref, reg_ref, cls_ref, tot_ref):
    total = jnp.float32(0.0)
    for l in range(3):
        cls_l = (sums_ref[l] + sums_ref[3 + l]) * (1.0 / _CLS_CNT)
        reg_l = jnp.sum(regp_ref[l]) * (1.0 / _REG_CNT)
        reg_ref[l] = reg_l
        cls_ref[l] = cls_l
        total = total + cls_l * _CLS_W + reg_l * _REG_W
    reg_ref[3] = 0.0
    cls_ref[3] = 0.0
    tot_ref[0] = total


def _run_encode_argmin(ego_t, anchor):
    return pl.pallas_call(
        _encode_argmin_body,
        out_shape=(
            jax.ShapeDtypeStruct((_D, _T, _B), jnp.float32),
            jax.ShapeDtypeStruct((_T, _B), jnp.int32),
            jax.ShapeDtypeStruct((_T, _B), jnp.int32),
        ),
        in_specs=[
            pl.BlockSpec(memory_space=pltpu.VMEM),
            pl.BlockSpec(memory_space=pltpu.SMEM),
        ],
        out_specs=(
            pl.BlockSpec(memory_space=pltpu.VMEM),
            pl.BlockSpec(memory_space=pltpu.VMEM),
            pl.BlockSpec(memory_space=pltpu.VMEM),
        ),
    )(ego_t, anchor)


def _run_dense_cls(cls0, cls1, cls2, bestn):
    blk = pl.BlockSpec((1, _T, _B), lambda n: (n, 0, 0))
    return pl.pallas_call(
        _dense_cls_body,
        grid=(_N,),
        out_shape=jax.ShapeDtypeStruct((8,), jnp.float32),
        in_specs=[blk, blk, blk, pl.BlockSpec((_T, _B), lambda n: (0, 0))],
        out_specs=pl.BlockSpec(memory_space=pltpu.SMEM),
    )(cls0, cls1, cls2, bestn)


def _run_sc_gather(reg0, reg1, reg2, idxpb, gt):
    mesh = plsc.VectorSubcoreMesh(core_axis_name="c", subcore_axis_name="s",
                                  num_cores=_NC)
    f32 = jnp.float32
    kern = functools.partial(
        pl.kernel,
        mesh=mesh,
        out_type=jax.ShapeDtypeStruct((3 * _NW * 16,), f32),
        scratch_types=(
            [pltpu.VMEM((_HALF,), jnp.int32)] * 2
            + [pltpu.VMEM((_HALF,), jnp.int32)] * 14
            + [pltpu.VMEM((_CH,), f32)] * _D
            + [pltpu.VMEM((_CH,), f32)] * 21
            + [pltpu.VMEM((16,), f32)]
            + [pltpu.SemaphoreType.DMA] * 23
        ),
    )(_sc_gather_body)
    return kern(reg0, reg1, reg2, idxpb, gt)


def _run_finalize(regp, sums):
    return pl.pallas_call(
        _finalize_body,
        out_shape=(
            jax.ShapeDtypeStruct((4,), jnp.float32),
            jax.ShapeDtypeStruct((4,), jnp.float32),
            jax.ShapeDtypeStruct((1,), jnp.float32),
        ),
        in_specs=[
            pl.BlockSpec(memory_space=pltpu.VMEM),
            pl.BlockSpec(memory_space=pltpu.SMEM),
        ],
        out_specs=(
            pl.BlockSpec(memory_space=pltpu.SMEM),
            pl.BlockSpec(memory_space=pltpu.SMEM),
            pl.BlockSpec(memory_space=pltpu.SMEM),
        ),
    )(regp, sums)


def kernel(diff_poses_reg_0, diff_poses_reg_1, diff_poses_reg_2,
           diff_poses_cls_0, diff_poses_cls_1, diff_poses_cls_2,
           ego_past_to_ego_curr, anchor):
    ego_t = jnp.transpose(ego_past_to_ego_curr, (1, 2, 3, 0)).reshape(_T, 16, _B)
    cls_t = tuple(jnp.transpose(c, (2, 1, 0))
                  for c in (diff_poses_cls_0, diff_poses_cls_1, diff_poses_cls_2))
    anchor2 = jnp.transpose(anchor.reshape(_N, 3), (1, 0))
    gt, idxpb, bestn = _run_encode_argmin(ego_t, anchor2)

    reg_flat = tuple(jnp.transpose(r, (2, 3, 1, 0)).reshape(_N * _D * _Q)
                     for r in (diff_poses_reg_0, diff_poses_reg_1, diff_poses_reg_2))
    regp = _run_sc_gather(*reg_flat, idxpb.reshape(_Q), gt.reshape(_D * _Q))
    sums = _run_dense_cls(*cls_t, bestn)

    reg4, cls4, tot = _run_finalize(regp.reshape(3, _NW, 16), sums)
    return reg4[:3], cls4[:3], tot[0]


# speedup vs baseline: 13.4589x; 1.0414x over previous
import functools

import jax
import jax.numpy as jnp
from jax import lax
from jax.experimental import pallas as pl
from jax.experimental.pallas import tpu as pltpu
from jax.experimental.pallas import tpu_sc as plsc

_B, _T, _N, _D = 128, 64, 20, 7
_Q = _T * _B
_CLS_W, _REG_W = 10.0, 8.0
_CLS_CNT = _B * (_T - 1) * _N
_REG_CNT = _B * (_T - 1) * _D

_NC, _NS = 2, 16
_NW = _NC * _NS
_CH = _Q // _NW
_HALF = _CH // 2


def _sgn(v):
    return jnp.where(v >= 0, 1.0, -1.0)


def _encode_argmin_body(ego_ref, anchor_ref, gt_ref, idxpb_ref, bestn_ref):
    e = [ego_ref[:, k, :] for k in range(12)]
    m00, m01, m02, tx = e[0], e[1], e[2], e[3]
    m10, m11, m12, ty = e[4], e[5], e[6], e[7]
    m20, m21, m22, tz = e[8], e[9], e[10], e[11]
    eps = 1e-12
    qw = 0.5 * jnp.sqrt(jnp.maximum(1.0 + m00 + m11 + m22, eps))
    qx = _sgn(m21 - m12) * 0.5 * jnp.sqrt(jnp.maximum(1.0 + m00 - m11 - m22, eps))
    qy = _sgn(m02 - m20) * 0.5 * jnp.sqrt(jnp.maximum(1.0 + m11 - m00 - m22, eps))
    qz = _sgn(m10 - m01) * 0.5 * jnp.sqrt(jnp.maximum(1.0 + m22 - m00 - m11, eps))
    comps = (tx, ty, tz, qw, qx, qy, qz)
    for c in range(_D):
        gt_ref[c] = comps[c]
    bestd = jnp.full((_T, _B), jnp.inf, jnp.float32)
    bestn = jnp.zeros((_T, _B), jnp.int32)
    for n in range(_N):
        ax = anchor_ref[0, n]
        ay = anchor_ref[1, n]
        az = anchor_ref[2, n]
        d = (tx - ax) ** 2 + (ty - ay) ** 2 + (tz - az) ** 2
        upd = d < bestd
        bestd = jnp.where(upd, d, bestd)
        bestn = jnp.where(upd, n, bestn)
    qv = (lax.broadcasted_iota(jnp.int32, (_T, _B), 0) * _B
          + lax.broadcasted_iota(jnp.int32, (_T, _B), 1))
    idxpb_ref[...] = bestn * (_D * _Q) + qv
    bestn_ref[...] = bestn


def _dense_cls_body(cls0_ref, cls1_ref, cls2_ref, bestn_ref, sums_ref):
    n = pl.program_id(0)

    @pl.when(n == 0)
    def _init():
        for l in range(8):
            sums_ref[l] = 0.0

    tmask = lax.broadcasted_iota(jnp.int32, (1, _T, _B), 1) >= 1
    sel = tmask & (bestn_ref[...][None] == n)
    for l, ref in enumerate((cls0_ref, cls1_ref, cls2_ref)):
        x = ref[...]
        u = jnp.exp(-jnp.abs(x))
        r = 1.0 / (1.0 + u)
        p = jnp.where(x >= 0, r, u * r)
        sp = jnp.maximum(x, 0.0) + jnp.log1p(u)
        l0 = sp * (0.75 * p * p)
        omp = 1.0 - p
        corr = (sp - x) * (0.25 * omp * omp) - l0
        sums_ref[l] += jnp.sum(jnp.where(tmask, l0, 0.0))
        sums_ref[3 + l] += jnp.sum(jnp.where(sel, corr, 0.0))


def _sc_gather_body(reg0, reg1, reg2, idxpb_hbm, gt_hbm, regp_out, *s):
    ipb_a, ipb_b = s[0], s[1]
    idx7 = [[s[2 + d * 2 + h] for h in range(2)] for d in range(_D)]
    gtv = [s[16 + d] for d in range(_D)]
    gbuf = [s[23 + i] for i in range(21)]
    accv = s[44]
    semi = s[45]
    semg = s[46]
    sems = [s[47 + i] for i in range(21)]
    regs = (reg0, reg1, reg2)
    wid = lax.axis_index("s") * _NC + lax.axis_index("c")
    base = wid * _CH
    idx_copies = [
        pltpu.async_copy(idxpb_hbm.at[pl.ds(base, _HALF)], ipb_a, semi),
        pltpu.async_copy(idxpb_hbm.at[pl.ds(base + _HALF, _HALF)], ipb_b, semi),
    ]
    gt_copies = [
        pltpu.async_copy(gt_hbm.at[pl.ds(d * _Q + base, _CH)], gtv[d], semg)
        for d in range(_D)
    ]
    for cp in idx_copies:
        cp.wait()

    def _build_chunk(i, carry):
        s16 = pl.ds(i * 16, 16)
        for h, iv in enumerate((ipb_a, ipb_b)):
            v = iv[s16]
            for d in range(_D):
                idx7[d][h][s16] = v + (d * _Q)
        return carry

    lax.fori_loop(0, _HALF // 16, _build_chunk, 0)
    steps = [(l, d) for l in range(3) for d in range(_D)]
    pend = []
    for si, (l, d) in enumerate(steps):
        pend.append([pltpu.async_copy(regs[l].at[idx7[d][h]],
                                      gbuf[si].at[pl.ds(h * _HALF, _HALF)],
                                      sems[si])
                     for h in range(2)])
    for cp in gt_copies:
        cp.wait()
    zf = jnp.where(wid == 0, jnp.float32(0.0), jnp.float32(1.0))
    one = jnp.float32(1.0)
    acc = jnp.zeros((16,), jnp.float32)
    for si, (l, d) in enumerate(steps):
        for cp in pend[si]:
            cp.wait()
        bb = gbuf[si]
        gv = gtv[d]

        def _chunk(i, a, bb=bb, gv=gv):
            s16 = pl.ds(i * 16, 16)
            diff = jnp.abs(bb[s16] - gv[s16])
            return a + diff * jnp.where(i < 8, zf, one)

        acc = lax.fori_loop(0, _CH // 16, _chunk, acc)
        if d == _D - 1:
            accv[...] = acc
            pltpu.sync_copy(accv, regp_out.at[pl.ds((l * _NW + wid) * 16, 16)])
            acc = jnp.zeros((16,), jnp.float32)


def _finalize_body(regp_ref, sums_ref, reg_ref, cls_ref, tot_ref):
    total = jnp.float32(0.0)
    for l in range(3):
        cls_l = (sums_ref[l] + sums_ref[3 + l]) * (1.0 / _CLS_CNT)
        reg_l = jnp.sum(regp_ref[pl.ds(l * _NW * 16, _NW * 16)]) * (1.0 / _REG_CNT)
        reg_ref[l] = reg_l
        cls_ref[l] = cls_l
        total = total + cls_l * _CLS_W + reg_l * _REG_W
    reg_ref[3] = 0.0
    cls_ref[3] = 0.0
    tot_ref[0] = total


def _run_encode_argmin(ego_t, anchor):
    return pl.pallas_call(
        _encode_argmin_body,
        out_shape=(
            jax.ShapeDtypeStruct((_D, _T, _B), jnp.float32),
            jax.ShapeDtypeStruct((_T, _B), jnp.int32),
            jax.ShapeDtypeStruct((_T, _B), jnp.int32),
        ),
        in_specs=[
            pl.BlockSpec(memory_space=pltpu.VMEM),
            pl.BlockSpec(memory_space=pltpu.SMEM),
        ],
        out_specs=(
            pl.BlockSpec(memory_space=pltpu.VMEM),
            pl.BlockSpec(memory_space=pltpu.VMEM),
            pl.BlockSpec(memory_space=pltpu.VMEM),
        ),
    )(ego_t, anchor)


def _run_dense_cls(cls0, cls1, cls2, bestn):
    blk = pl.BlockSpec((1, _T, _B), lambda n: (n, 0, 0))
    return pl.pallas_call(
        _dense_cls_body,
        grid=(_N,),
        out_shape=jax.ShapeDtypeStruct((8,), jnp.float32),
        in_specs=[blk, blk, blk, pl.BlockSpec((_T, _B), lambda n: (0, 0))],
        out_specs=pl.BlockSpec(memory_space=pltpu.SMEM),
    )(cls0, cls1, cls2, bestn)


def _run_sc_gather(reg0, reg1, reg2, idxpb, gt):
    mesh = plsc.VectorSubcoreMesh(core_axis_name="c", subcore_axis_name="s",
                                  num_cores=_NC)
    f32 = jnp.float32
    kern = functools.partial(
        pl.kernel,
        mesh=mesh,
        out_type=jax.ShapeDtypeStruct((3 * _NW * 16,), f32),
        scratch_types=(
            [pltpu.VMEM((_HALF,), jnp.int32)] * 2
            + [pltpu.VMEM((_HALF,), jnp.int32)] * 14
            + [pltpu.VMEM((_CH,), f32)] * _D
            + [pltpu.VMEM((_CH,), f32)] * 21
            + [pltpu.VMEM((16,), f32)]
            + [pltpu.SemaphoreType.DMA] * 23
        ),
    )(_sc_gather_body)
    return kern(reg0, reg1, reg2, idxpb, gt)


def _run_finalize(regp, sums):
    return pl.pallas_call(
        _finalize_body,
        out_shape=(
            jax.ShapeDtypeStruct((4,), jnp.float32),
            jax.ShapeDtypeStruct((4,), jnp.float32),
            jax.ShapeDtypeStruct((1,), jnp.float32),
        ),
        in_specs=[
            pl.BlockSpec(memory_space=pltpu.VMEM),
            pl.BlockSpec(memory_space=pltpu.SMEM),
        ],
        out_specs=(
            pl.BlockSpec(memory_space=pltpu.SMEM),
            pl.BlockSpec(memory_space=pltpu.SMEM),
            pl.BlockSpec(memory_space=pltpu.SMEM),
        ),
    )(regp, sums)


def kernel(diff_poses_reg_0, diff_poses_reg_1, diff_poses_reg_2,
           diff_poses_cls_0, diff_poses_cls_1, diff_poses_cls_2,
           ego_past_to_ego_curr, anchor):
    ego_t = jnp.transpose(ego_past_to_ego_curr, (1, 2, 3, 0)).reshape(_T, 16, _B)
    cls_t = tuple(jnp.transpose(c, (2, 1, 0))
                  for c in (diff_poses_cls_0, diff_poses_cls_1, diff_poses_cls_2))
    anchor2 = jnp.transpose(anchor.reshape(_N, 3), (1, 0))
    gt, idxpb, bestn = _run_encode_argmin(ego_t, anchor2)

    reg_flat = tuple(jnp.transpose(r, (2, 3, 1, 0)).reshape(_N * _D * _Q)
                     for r in (diff_poses_reg_0, diff_poses_reg_1, diff_poses_reg_2))
    regp = _run_sc_gather(*reg_flat, idxpb.reshape(_Q), gt.reshape(_D * _Q))
    sums = _run_dense_cls(*cls_t, bestn)

    reg4, cls4, tot = _run_finalize(regp, sums)
    return reg4[:3], cls4[:3], tot[0]
